# baseline (speedup 1.0000x reference)
import functools

import jax
import jax.numpy as jnp
from jax import lax
from jax.experimental import pallas as pl
from jax.experimental.pallas import tpu as pltpu
from jax.experimental.pallas import tpu_sc as plsc

CHUNK = 256
K = 2
NBUF = 3


@functools.lru_cache(maxsize=None)
def _make_kernel(B, D):
    info = plsc.get_sparse_core_info()
    NC, NS = info.num_cores, info.num_subcores
    NW = NC * NS
    b_per_w = B // NW
    n_chunks = b_per_w // CHUNK
    T = n_chunks // K
    assert B == NW * T * K * CHUNK and T >= 6 and (T - 5) % NBUF == 0

    mesh = plsc.VectorSubcoreMesh(core_axis_name="c", subcore_axis_name="s")

    @functools.partial(
        pl.kernel,
        out_type=jax.ShapeDtypeStruct((B, D), jnp.float32),
        mesh=mesh,
        compiler_params=pltpu.CompilerParams(use_tc_tiling_on_sc=False),
        scratch_types=[
            pltpu.VMEM((n_chunks, CHUNK), jnp.int32),
            pltpu.VMEM((NBUF, K * CHUNK, D), jnp.float32),
            pltpu.SemaphoreType.DMA,
            pltpu.SemaphoreType.DMA,
            pltpu.SemaphoreType.DMA,
            pltpu.SemaphoreType.DMA,
            pltpu.SemaphoreType.DMA,
            pltpu.SemaphoreType.DMA,
        ],
    )
    def gather_kernel(
        x_hbm, table_hbm, out_hbm, idx_v, rows_v, g0, g1, g2, s0, s1, s2
    ):
        gsem = (g0, g1, g2)
        ssem = (s0, s1, s2)
        wid = lax.axis_index("s") * NC + lax.axis_index("c")
        row0 = wid * b_per_w

        pltpu.sync_copy(x_hbm.at[wid], idx_v)

        def gathers(t, p):
            return [
                pltpu.make_async_copy(
                    table_hbm.at[idx_v.at[t * K + j]],
                    rows_v.at[p, pl.ds(j * CHUNK, CHUNK)],
                    gsem[p],
                )
                for j in range(K)
            ]

        def stores(t, p):
            return [
                pltpu.make_async_copy(
                    rows_v.at[p],
                    out_hbm.at[pl.ds(row0 + t * K * CHUNK, K * CHUNK)],
                    ssem[p],
                )
            ]

        def step(u, p, drain_prev=True, fire_next=True):
            for d in gathers(u, p):
                d.wait()
            for d in stores(u, p):
                d.start()
            if drain_prev:
                for d in stores(u - 1, (p + 2) % NBUF):
                    d.wait()
            if fire_next:
                for d in gathers(u + 2, (p + 2) % NBUF):
                    d.start()

        for t in range(2):
            for d in gathers(t, t):
                d.start()
        step(0, 0, drain_prev=False)

        def body(i, _):
            u0 = NBUF * i + 1
            for dp in range(NBUF):
                step(u0 + dp, (1 + dp) % NBUF)
            return _

        lax.fori_loop(0, (T - 5) // NBUF, body, None)

        for u in range(T - 4, T):
            step(u, u % NBUF, fire_next=(u + 2 < T))
        for d in stores(T - 1, (T - 1) % NBUF):
            d.wait()

    return gather_kernel


def kernel(x, lut_weight):
    B, H = x.shape
    D = lut_weight.shape[1]
    info = plsc.get_sparse_core_info()
    NW = info.num_cores * info.num_subcores
    n_chunks = (B * H) // (NW * CHUNK)
    idx = x.astype(jnp.int32).reshape(NW, n_chunks, CHUNK)
    out = _make_kernel(B * H, D)(idx, lut_weight)
    return out.reshape(B, H, D)


# speedup vs baseline: 1.0077x; 1.0077x over previous
import functools

import jax
import jax.numpy as jnp
from jax import lax
from jax.experimental import pallas as pl
from jax.experimental.pallas import tpu as pltpu
from jax.experimental.pallas import tpu_sc as plsc

CHUNK = 512
K = 1
NBUF = 3


@functools.lru_cache(maxsize=None)
def _make_kernel(B, D):
    info = plsc.get_sparse_core_info()
    NC, NS = info.num_cores, info.num_subcores
    NW = NC * NS
    b_per_w = B // NW
    n_chunks = b_per_w // CHUNK
    T = n_chunks // K
    assert B == NW * T * K * CHUNK and T >= 6 and (T - 5) % NBUF == 0

    mesh = plsc.VectorSubcoreMesh(core_axis_name="c", subcore_axis_name="s")

    @functools.partial(
        pl.kernel,
        out_type=jax.ShapeDtypeStruct((B, D), jnp.float32),
        mesh=mesh,
        compiler_params=pltpu.CompilerParams(use_tc_tiling_on_sc=False),
        scratch_types=[
            pltpu.VMEM((n_chunks, CHUNK), jnp.int32),
            pltpu.VMEM((NBUF, K * CHUNK, D), jnp.float32),
            pltpu.SemaphoreType.DMA,
            pltpu.SemaphoreType.DMA,
            pltpu.SemaphoreType.DMA,
            pltpu.SemaphoreType.DMA,
            pltpu.SemaphoreType.DMA,
            pltpu.SemaphoreType.DMA,
        ],
    )
    def gather_kernel(
        x_hbm, table_hbm, out_hbm, idx_v, rows_v, g0, g1, g2, s0, s1, s2
    ):
        gsem = (g0, g1, g2)
        ssem = (s0, s1, s2)
        wid = lax.axis_index("s") * NC + lax.axis_index("c")
        row0 = wid * b_per_w

        pltpu.sync_copy(x_hbm.at[wid], idx_v)

        def gathers(t, p):
            return [
                pltpu.make_async_copy(
                    table_hbm.at[idx_v.at[t * K + j]],
                    rows_v.at[p, pl.ds(j * CHUNK, CHUNK)],
                    gsem[p],
                )
                for j in range(K)
            ]

        def stores(t, p):
            return [
                pltpu.make_async_copy(
                    rows_v.at[p],
                    out_hbm.at[pl.ds(row0 + t * K * CHUNK, K * CHUNK)],
                    ssem[p],
                )
            ]

        def step(u, p, drain_prev=True, fire_next=True):
            for d in gathers(u, p):
                d.wait()
            for d in stores(u, p):
                d.start()
            if drain_prev:
                for d in stores(u - 1, (p + 2) % NBUF):
                    d.wait()
            if fire_next:
                for d in gathers(u + 2, (p + 2) % NBUF):
                    d.start()

        for t in range(2):
            for d in gathers(t, t):
                d.start()
        step(0, 0, drain_prev=False)

        def body(i, _):
            u0 = NBUF * i + 1
            for dp in range(NBUF):
                step(u0 + dp, (1 + dp) % NBUF)
            return _

        lax.fori_loop(0, (T - 5) // NBUF, body, None)

        for u in range(T - 4, T):
            step(u, u % NBUF, fire_next=(u + 2 < T))
        for d in stores(T - 1, (T - 1) % NBUF):
            d.wait()

    return gather_kernel


def kernel(x, lut_weight):
    B, H = x.shape
    D = lut_weight.shape[1]
    info = plsc.get_sparse_core_info()
    NW = info.num_cores * info.num_subcores
    n_chunks = (B * H) // (NW * CHUNK)
    idx = x.astype(jnp.int32).reshape(NW, n_chunks, CHUNK)
    out = _make_kernel(B * H, D)(idx, lut_weight)
    return out.reshape(B, H, D)
